# Initial kernel scaffold; baseline (speedup 1.0000x reference)
import jax
import jax.numpy as jnp
from jax.experimental import pallas as pl


def kernel(x, edge_index, W_neigh0, W_self0, b0, W_neigh1, W_self1, b1):
    raise NotImplementedError("write your pallas kernel here")


# speedup vs baseline: 4.8831x; 4.8831x over previous
import functools

import jax
import jax.numpy as jnp
from jax import lax
from jax.experimental import pallas as pl
from jax.experimental.pallas import tpu as pltpu
from jax.experimental.pallas import tpu_sc as plsc

_N = 10000
_E = 320000
_D = 128
_H = 128
_C = 64

_NC = 2
_NS = 16
_NW = _NC * _NS
_L = 16

_CHUNK = 128
_NCHUNK = -(-_E // (_NW * _CHUNK))
_EPAD = _NW * _NCHUNK * _CHUNK
_NPAD = 10240
_RPS = _NPAD // _NS

_ROWBLK = 512


def _prep_edges(edge_index):
    src = edge_index[0]
    dst = edge_index[1]
    pad = _EPAD - _E
    src = jnp.concatenate([src, jnp.full((pad,), _N, jnp.int32)])
    dst = jnp.concatenate([dst, jnp.full((pad,), _N, jnp.int32)])
    return (src.reshape(_NW, _NCHUNK, _CHUNK),
            dst.reshape(_NW, _NCHUNK, _CHUNK))


def _sc_aggregate(table, src3, dst3, zfeat, feat):
    mesh = plsc.VectorSubcoreMesh(core_axis_name="c", subcore_axis_name="s")

    @functools.partial(
        pl.kernel,
        out_type=[jax.ShapeDtypeStruct((_NC, _NPAD, feat), jnp.float32)],
        mesh=mesh,
        scratch_types=[
            pltpu.VMEM((_NCHUNK, _CHUNK), jnp.int32),
            pltpu.VMEM((_NCHUNK, _CHUNK), jnp.int32),
            pltpu.VMEM((_CHUNK, feat), jnp.float32),
            pltpu.VMEM_SHARED((_NPAD, feat), jnp.float32),
            pltpu.SemaphoreType.DMA,
        ])
    def kern(t_hbm, src_hbm, dst_hbm, zf_hbm, acc_out,
             src_v, dst_v, rows_v, acc_s, sem):
        cid = lax.axis_index("c")
        sid = lax.axis_index("s")
        wid = sid * _NC + cid
        r0 = sid * _RPS
        pltpu.sync_copy(zf_hbm.at[pl.ds(r0, _RPS)],
                        acc_s.at[pl.ds(r0, _RPS)])
        pltpu.sync_copy(src_hbm.at[wid], src_v)
        pltpu.sync_copy(dst_hbm.at[wid], dst_v)
        plsc.subcore_barrier()

        @pl.loop(0, _NCHUNK)
        def _(j):
            pltpu.async_copy(t_hbm.at[src_v.at[j]], rows_v, sem).wait()
            pltpu.sync_copy(rows_v, acc_s.at[dst_v.at[j]], add=True)

        plsc.subcore_barrier()
        pltpu.sync_copy(acc_s.at[pl.ds(r0, _RPS)],
                        acc_out.at[cid, pl.ds(r0, _RPS)])

    return kern(table, src3, dst3, zfeat)[0]


def _sc_degree(dst3, zdeg, ones):
    mesh = plsc.VectorSubcoreMesh(core_axis_name="c", subcore_axis_name="s")

    @functools.partial(
        pl.kernel,
        out_type=[jax.ShapeDtypeStruct((_NC, _NPAD, _D), jnp.float32)],
        mesh=mesh,
        scratch_types=[
            pltpu.VMEM((_NCHUNK, _CHUNK), jnp.int32),
            pltpu.VMEM((_CHUNK, _D), jnp.float32),
            pltpu.VMEM_SHARED((_NPAD, _D), jnp.float32),
        ])
    def kern(dst_hbm, zd_hbm, ones_hbm, deg_out, dst_v, ones_v, deg_s):
        cid = lax.axis_index("c")
        sid = lax.axis_index("s")
        wid = sid * _NC + cid
        r0 = sid * _RPS
        pltpu.sync_copy(zd_hbm.at[pl.ds(r0, _RPS)],
                        deg_s.at[pl.ds(r0, _RPS)])
        pltpu.sync_copy(ones_hbm, ones_v)
        pltpu.sync_copy(dst_hbm.at[wid], dst_v)
        plsc.subcore_barrier()

        @pl.loop(0, _NCHUNK)
        def _(j):
            pltpu.sync_copy(ones_v, deg_s.at[dst_v.at[j]], add=True)

        plsc.subcore_barrier()
        pltpu.sync_copy(deg_s.at[pl.ds(r0, _RPS)],
                        deg_out.at[cid, pl.ds(r0, _RPS)])

    return kern(dst3, zdeg, ones)[0]


def _tc_lin2(x, wa, wb, b):
    m, k = x.shape
    na, nb = wa.shape[1], wb.shape[1]
    grid = m // _ROWBLK

    def body(x_ref, wa_ref, wb_ref, b_ref, oa_ref, ob_ref):
        xv = x_ref[...]
        oa_ref[...] = jnp.dot(xv, wa_ref[...],
                              preferred_element_type=jnp.float32)
        ob_ref[...] = jnp.dot(xv, wb_ref[...],
                              preferred_element_type=jnp.float32) + b_ref[...]

    return pl.pallas_call(
        body,
        grid=(grid,),
        in_specs=[
            pl.BlockSpec((_ROWBLK, k), lambda i: (i, 0)),
            pl.BlockSpec((k, na), lambda i: (0, 0)),
            pl.BlockSpec((k, nb), lambda i: (0, 0)),
            pl.BlockSpec((1, nb), lambda i: (0, 0)),
        ],
        out_specs=[
            pl.BlockSpec((_ROWBLK, na), lambda i: (i, 0)),
            pl.BlockSpec((_ROWBLK, nb), lambda i: (i, 0)),
        ],
        out_shape=[
            jax.ShapeDtypeStruct((m, na), jnp.float32),
            jax.ShapeDtypeStruct((m, nb), jnp.float32),
        ],
    )(x, wa, wb, b)


def _tc_mid(acc0, degacc, s0x, wn1, ws1, b1):
    grid = _NPAD // _ROWBLK

    def body(a_ref, d_ref, s_ref, wn_ref, ws_ref, b_ref, p_ref, s1_ref):
        av = a_ref[...]
        dv = d_ref[...]
        deg = jnp.maximum(dv[0, :, 0:1] + dv[1, :, 0:1], 1.0)
        h = jnp.maximum((av[0] + av[1]) / deg + s_ref[...], 0.0)
        p_ref[...] = jnp.dot(h, wn_ref[...],
                             preferred_element_type=jnp.float32)
        s1_ref[...] = jnp.dot(h, ws_ref[...],
                              preferred_element_type=jnp.float32) + b_ref[...]

    return pl.pallas_call(
        body,
        grid=(grid,),
        in_specs=[
            pl.BlockSpec((_NC, _ROWBLK, _H), lambda i: (0, i, 0)),
            pl.BlockSpec((_NC, _ROWBLK, _D), lambda i: (0, i, 0)),
            pl.BlockSpec((_ROWBLK, _H), lambda i: (i, 0)),
            pl.BlockSpec((_H, _D), lambda i: (0, 0)),
            pl.BlockSpec((_H, _C), lambda i: (0, 0)),
            pl.BlockSpec((1, _C), lambda i: (0, 0)),
        ],
        out_specs=[
            pl.BlockSpec((_ROWBLK, _D), lambda i: (i, 0)),
            pl.BlockSpec((_ROWBLK, _C), lambda i: (i, 0)),
        ],
        out_shape=[
            jax.ShapeDtypeStruct((_NPAD, _D), jnp.float32),
            jax.ShapeDtypeStruct((_NPAD, _C), jnp.float32),
        ],
    )(acc0, degacc, s0x, wn1, ws1, b1)


def _tc_out(acc1, degacc, s1):
    grid = _NPAD // _ROWBLK

    def body(a_ref, d_ref, s_ref, o_ref):
        av = a_ref[...]
        dv = d_ref[...]
        deg = jnp.maximum(dv[0, :, 0:1] + dv[1, :, 0:1], 1.0)
        o_ref[...] = (av[0] + av[1])[:, :_C] / deg + s_ref[...]

    return pl.pallas_call(
        body,
        grid=(grid,),
        in_specs=[
            pl.BlockSpec((_NC, _ROWBLK, _D), lambda i: (0, i, 0)),
            pl.BlockSpec((_NC, _ROWBLK, _D), lambda i: (0, i, 0)),
            pl.BlockSpec((_ROWBLK, _C), lambda i: (i, 0)),
        ],
        out_specs=pl.BlockSpec((_ROWBLK, _C), lambda i: (i, 0)),
        out_shape=jax.ShapeDtypeStruct((_NPAD, _C), jnp.float32),
    )(acc1, degacc, s1)


def kernel(x, edge_index, W_neigh0, W_self0, b0, W_neigh1, W_self1, b1):
    x_pad = jnp.pad(x, ((0, _NPAD - _N), (0, 0)))
    src3, dst3 = _prep_edges(edge_index)
    zD = jnp.zeros((_NPAD, _D), jnp.float32)
    wn1_pad = jnp.pad(W_neigh1, ((0, 0), (0, _D - _C)))

    degacc = _sc_degree(dst3, zD, jnp.ones((_CHUNK, _D), jnp.float32))
    q0, s0x = _tc_lin2(x_pad, W_neigh0, W_self0, b0.reshape(1, _H))
    acc0 = _sc_aggregate(q0, src3, dst3, zD, _D)
    p, s1 = _tc_mid(acc0, degacc, s0x, wn1_pad, W_self1, b1.reshape(1, _C))
    acc1 = _sc_aggregate(p, src3, dst3, zD, _D)
    out_full = _tc_out(acc1, degacc, s1)
    return out_full[:_N]
